# Initial kernel scaffold; baseline (speedup 1.0000x reference)
import jax
import jax.numpy as jnp
from jax.experimental import pallas as pl


def kernel(features, edge_index, W1, b1, W2, b2, W3, b3):
    raise NotImplementedError("write your pallas kernel here")


# speedup vs baseline: 5.4399x; 5.4399x over previous
import functools

import jax
import jax.numpy as jnp
from jax import lax
from jax.experimental import pallas as pl
from jax.experimental.pallas import tpu as pltpu, tpu_sc as plsc

N_NODES = 10000
N_PAD = 10240
N_EDGES = 320000
EROW = 128
EROWS = 2560
PAD_IDX = N_PAD - 1
NC, NS = 2, 16
ROWS_PER_TILE = EROWS // NS
STRIPE = N_PAD // NS
BM = 512
GRID_M = N_PAD // BM

_mesh = plsc.VectorSubcoreMesh(core_axis_name="c", subcore_axis_name="s",
                               num_cores=NC, num_subcores=NS)
_sc_params = pltpu.CompilerParams(use_tc_tiling_on_sc=False)


def _deg_body(edges, ones_h, zeros_h, hist, idx_v, ones_v, zbuf, hist_sp):
    c = lax.axis_index("c")
    s = lax.axis_index("s")
    base = s * ROWS_PER_TILE
    pltpu.sync_copy(edges.at[1 - c].at[pl.ds(base, ROWS_PER_TILE)], idx_v)
    pltpu.sync_copy(ones_h, ones_v)
    pltpu.sync_copy(zeros_h, zbuf)
    for k in range(STRIPE // EROW):
        pltpu.sync_copy(zbuf, hist_sp.at[pl.ds(s * STRIPE + k * EROW, EROW)])
    plsc.subcore_barrier()

    def body(r, carry):
        pltpu.sync_copy(ones_v, hist_sp.at[idx_v.at[r]], add=True)
        return carry

    lax.fori_loop(0, ROWS_PER_TILE, body, 0)
    plsc.subcore_barrier()
    pltpu.sync_copy(hist_sp.at[pl.ds(s * STRIPE, STRIPE)],
                    hist.at[c].at[pl.ds(s * STRIPE, STRIPE)])


_deg_kernel = pl.kernel(
    _deg_body,
    out_type=jax.ShapeDtypeStruct((NC, N_PAD, 16), jnp.float32),
    mesh=_mesh,
    compiler_params=_sc_params,
    scratch_types=[
        pltpu.VMEM((ROWS_PER_TILE, EROW), jnp.int32),
        pltpu.VMEM((EROW, 16), jnp.float32),
        pltpu.VMEM((EROW, 16), jnp.float32),
        pltpu.VMEM_SHARED((N_PAD, 16), jnp.float32),
    ],
)


IDX_CHUNK = 32
N_CHUNKS = ROWS_PER_TILE // IDX_CHUNK


def _agg_body(fc, table, edges, zeros_h, agg,
              src_v, dst_v, gbuf, agg_sp, gsem):
    c = lax.axis_index("c")
    s = lax.axis_index("s")
    base = s * ROWS_PER_TILE
    pltpu.sync_copy(zeros_h, gbuf)
    for k in range(STRIPE // EROW):
        pltpu.sync_copy(gbuf, agg_sp.at[pl.ds(s * STRIPE + k * EROW, EROW)])
    plsc.subcore_barrier()

    def row(r, carry):
        pltpu.async_copy(table.at[c].at[src_v.at[r]], gbuf, gsem).wait()
        pltpu.sync_copy(gbuf, agg_sp.at[dst_v.at[r]], add=True)
        return carry

    for ci in range(N_CHUNKS):
        pltpu.sync_copy(edges.at[0].at[pl.ds(base + ci * IDX_CHUNK, IDX_CHUNK)],
                        src_v)
        pltpu.sync_copy(edges.at[1].at[pl.ds(base + ci * IDX_CHUNK, IDX_CHUNK)],
                        dst_v)
        lax.fori_loop(0, IDX_CHUNK, row, 0)
    plsc.subcore_barrier()
    pltpu.sync_copy(agg_sp.at[pl.ds(s * STRIPE, STRIPE)],
                    agg.at[c].at[pl.ds(s * STRIPE, STRIPE)])


def _make_agg(fc):
    return pl.kernel(
        functools.partial(_agg_body, fc),
        out_type=jax.ShapeDtypeStruct((NC, N_PAD, fc), jnp.float32),
        mesh=_mesh,
        compiler_params=_sc_params,
        scratch_types=[
            pltpu.VMEM((IDX_CHUNK, EROW), jnp.int32),
            pltpu.VMEM((IDX_CHUNK, EROW), jnp.int32),
            pltpu.VMEM((EROW, fc), jnp.float32),
            pltpu.VMEM_SHARED((N_PAD, fc), jnp.float32),
            pltpu.SemaphoreType.DMA,
        ],
    )


_agg64 = _make_agg(64)
_agg128 = _make_agg(128)
_agg32 = _make_agg(32)


def _norm(col):
    return lax.rsqrt(jnp.maximum(col, 1.0))


def _dot(a, b):
    return lax.dot_general(a, b, (((1,), (0,)), ((), ())),
                           precision=lax.Precision.HIGHEST,
                           preferred_element_type=jnp.float32)


def _prep_body(x_ref, hist_ref, xs_ref):
    ns = _norm(hist_ref[1][:, :1])
    xs = x_ref[...] * ns
    xs_ref[0, :, :] = xs[:, :64]
    xs_ref[1, :, :] = xs[:, 64:]


_prep_kernel = pl.pallas_call(
    _prep_body,
    grid=(GRID_M,),
    in_specs=[
        pl.BlockSpec((BM, 128), lambda m: (m, 0)),
        pl.BlockSpec((NC, BM, 16), lambda m: (0, m, 0)),
    ],
    out_specs=pl.BlockSpec((NC, BM, 64), lambda m: (0, m, 0)),
    out_shape=jax.ShapeDtypeStruct((NC, N_PAD, 64), jnp.float32),
)


def _layer1_body(agg_ref, hist_ref, w_ref, b_ref, out_ref):
    a = jnp.concatenate([agg_ref[0], agg_ref[1]], axis=1)
    h = _dot(a, w_ref[...])
    nd = _norm(hist_ref[0][:, :1])
    ns = _norm(hist_ref[1][:, :1])
    g = ns * jnp.maximum(nd * h + b_ref[...], 0.0)
    out_ref[0, :, :] = g[:, :128]
    out_ref[1, :, :] = g[:, 128:]


_layer1_kernel = pl.pallas_call(
    _layer1_body,
    grid=(GRID_M,),
    in_specs=[
        pl.BlockSpec((NC, BM, 64), lambda m: (0, m, 0)),
        pl.BlockSpec((NC, BM, 16), lambda m: (0, m, 0)),
        pl.BlockSpec((128, 256), lambda m: (0, 0)),
        pl.BlockSpec((1, 256), lambda m: (0, 0)),
    ],
    out_specs=pl.BlockSpec((NC, BM, 128), lambda m: (0, m, 0)),
    out_shape=jax.ShapeDtypeStruct((NC, N_PAD, 128), jnp.float32),
)


def _layer2_body(agg_ref, hist_ref, w2_ref, b2_ref, w3_ref, out_ref):
    a = jnp.concatenate([agg_ref[0], agg_ref[1]], axis=1)
    h = _dot(a, w2_ref[...])
    nd = _norm(hist_ref[0][:, :1])
    ns = _norm(hist_ref[1][:, :1])
    g = ns * jnp.maximum(nd * h + b2_ref[...], 0.0)
    t = _dot(g, w3_ref[...])
    out_ref[0, :, :] = t[:, :32]
    out_ref[1, :, :] = t[:, 32:]


_layer2_kernel = pl.pallas_call(
    _layer2_body,
    grid=(GRID_M,),
    in_specs=[
        pl.BlockSpec((NC, BM, 128), lambda m: (0, m, 0)),
        pl.BlockSpec((NC, BM, 16), lambda m: (0, m, 0)),
        pl.BlockSpec((256, 256), lambda m: (0, 0)),
        pl.BlockSpec((1, 256), lambda m: (0, 0)),
        pl.BlockSpec((256, 64), lambda m: (0, 0)),
    ],
    out_specs=pl.BlockSpec((NC, BM, 32), lambda m: (0, m, 0)),
    out_shape=jax.ShapeDtypeStruct((NC, N_PAD, 32), jnp.float32),
)


def _final_body(agg_ref, hist_ref, b_ref, out_ref):
    a = jnp.concatenate([agg_ref[0], agg_ref[1]], axis=1)
    nd = _norm(hist_ref[0][:, :1])
    out_ref[...] = nd * a + b_ref[...]


_final_kernel = pl.pallas_call(
    _final_body,
    grid=(GRID_M,),
    in_specs=[
        pl.BlockSpec((NC, BM, 32), lambda m: (0, m, 0)),
        pl.BlockSpec((NC, BM, 16), lambda m: (0, m, 0)),
        pl.BlockSpec((1, 64), lambda m: (0, 0)),
    ],
    out_specs=pl.BlockSpec((BM, 64), lambda m: (m, 0)),
    out_shape=jax.ShapeDtypeStruct((N_PAD, 64), jnp.float32),
)


def kernel(features, edge_index, W1, b1, W2, b2, W3, b3):
    epad = jnp.full((2, EROWS * EROW - N_EDGES), PAD_IDX, jnp.int32)
    edges = jnp.concatenate([edge_index, epad], axis=1).reshape(2, EROWS, EROW)
    x = jnp.concatenate(
        [features, jnp.zeros((N_PAD - N_NODES, 128), jnp.float32)], axis=0)
    ones16 = jnp.ones((EROW, 16), jnp.float32)
    zeros16 = jnp.zeros((EROW, 16), jnp.float32)

    hist = _deg_kernel(edges, ones16, zeros16)
    xs = _prep_kernel(x, hist)
    a1 = _agg64(xs, edges, jnp.zeros((EROW, 64), jnp.float32))
    g1 = _layer1_kernel(a1, hist, W1, b1.reshape(1, -1))
    a2 = _agg128(g1, edges, jnp.zeros((EROW, 128), jnp.float32))
    t = _layer2_kernel(a2, hist, W2, b2.reshape(1, -1), W3)
    a3 = _agg32(t, edges, jnp.zeros((EROW, 32), jnp.float32))
    out = _final_kernel(a3, hist, b3.reshape(1, -1))
    return out[:N_NODES]


# speedup vs baseline: 6.7051x; 1.2326x over previous
import functools

import jax
import jax.numpy as jnp
from jax import lax
from jax.experimental import pallas as pl
from jax.experimental.pallas import tpu as pltpu, tpu_sc as plsc

N_NODES = 10000
N_PAD = 10240
N_EDGES = 320000
EROW = 128
EROWS = 2560
PAD_IDX = N_PAD - 1
NC, NS = 2, 16
ROWS_PER_TILE = EROWS // NS
STRIPE = N_PAD // NS
BM = 512
GRID_M = N_PAD // BM

_mesh = plsc.VectorSubcoreMesh(core_axis_name="c", subcore_axis_name="s",
                               num_cores=NC, num_subcores=NS)
_sc_params = pltpu.CompilerParams(use_tc_tiling_on_sc=False)


def _deg_body(edges, ones_h, zeros_h, hist, idx_v, ones_v, zbuf, hist_sp):
    c = lax.axis_index("c")
    s = lax.axis_index("s")
    base = s * ROWS_PER_TILE
    pltpu.sync_copy(edges.at[1 - c].at[pl.ds(base, ROWS_PER_TILE)], idx_v)
    pltpu.sync_copy(ones_h, ones_v)
    pltpu.sync_copy(zeros_h, zbuf)
    for k in range(STRIPE // EROW):
        pltpu.sync_copy(zbuf, hist_sp.at[pl.ds(s * STRIPE + k * EROW, EROW)])
    plsc.subcore_barrier()

    def body(r, carry):
        pltpu.sync_copy(ones_v, hist_sp.at[idx_v.at[r]], add=True)
        return carry

    lax.fori_loop(0, ROWS_PER_TILE, body, 0)
    plsc.subcore_barrier()
    pltpu.sync_copy(hist_sp.at[pl.ds(s * STRIPE, STRIPE)],
                    hist.at[c].at[pl.ds(s * STRIPE, STRIPE)])


_deg_kernel = pl.kernel(
    _deg_body,
    out_type=jax.ShapeDtypeStruct((NC, N_PAD, 16), jnp.float32),
    mesh=_mesh,
    compiler_params=_sc_params,
    scratch_types=[
        pltpu.VMEM((ROWS_PER_TILE, EROW), jnp.int32),
        pltpu.VMEM((EROW, 16), jnp.float32),
        pltpu.VMEM((EROW, 16), jnp.float32),
        pltpu.VMEM_SHARED((N_PAD, 16), jnp.float32),
    ],
)


IDX_CHUNK = 32
N_CHUNKS = ROWS_PER_TILE // IDX_CHUNK


def _agg_body(fc, table, edges, zeros_h, agg,
              src_v, dst_v, gbuf, agg_sp, gsem):
    c = lax.axis_index("c")
    s = lax.axis_index("s")
    base = s * ROWS_PER_TILE
    gbuf0, gbuf1 = gbuf
    sem0, sem1 = gsem
    pltpu.sync_copy(zeros_h, gbuf0)
    for k in range(STRIPE // EROW):
        pltpu.sync_copy(gbuf0, agg_sp.at[pl.ds(s * STRIPE + k * EROW, EROW)])
    plsc.subcore_barrier()

    tbl = table.at[c]
    for ci in range(N_CHUNKS):
        pltpu.sync_copy(edges.at[0].at[pl.ds(base + ci * IDX_CHUNK, IDX_CHUNK)],
                        src_v)
        pltpu.sync_copy(edges.at[1].at[pl.ds(base + ci * IDX_CHUNK, IDX_CHUNK)],
                        dst_v)
        pltpu.async_copy(tbl.at[src_v.at[0]], gbuf0, sem0)
        pltpu.async_copy(tbl.at[src_v.at[1]], gbuf1, sem1)

        def pair(k, carry):
            r = 2 * k
            pltpu.make_async_copy(tbl.at[src_v.at[r]], gbuf0, sem0).wait()
            pltpu.sync_copy(gbuf0, agg_sp.at[dst_v.at[r]], add=True)
            pltpu.async_copy(tbl.at[src_v.at[r + 2]], gbuf0, sem0)
            pltpu.make_async_copy(tbl.at[src_v.at[r + 1]], gbuf1, sem1).wait()
            pltpu.sync_copy(gbuf1, agg_sp.at[dst_v.at[r + 1]], add=True)
            pltpu.async_copy(tbl.at[src_v.at[r + 3]], gbuf1, sem1)
            return carry

        lax.fori_loop(0, IDX_CHUNK // 2 - 1, pair, 0)
        pltpu.make_async_copy(tbl.at[src_v.at[IDX_CHUNK - 2]], gbuf0,
                              sem0).wait()
        pltpu.sync_copy(gbuf0, agg_sp.at[dst_v.at[IDX_CHUNK - 2]], add=True)
        pltpu.make_async_copy(tbl.at[src_v.at[IDX_CHUNK - 1]], gbuf1,
                              sem1).wait()
        pltpu.sync_copy(gbuf1, agg_sp.at[dst_v.at[IDX_CHUNK - 1]], add=True)
    plsc.subcore_barrier()
    pltpu.sync_copy(agg_sp.at[pl.ds(s * STRIPE, STRIPE)],
                    agg.at[c].at[pl.ds(s * STRIPE, STRIPE)])


def _make_agg(fc):
    return pl.kernel(
        functools.partial(_agg_body, fc),
        out_type=jax.ShapeDtypeStruct((NC, N_PAD, fc), jnp.float32),
        mesh=_mesh,
        compiler_params=_sc_params,
        scratch_types=[
            pltpu.VMEM((IDX_CHUNK, EROW), jnp.int32),
            pltpu.VMEM((IDX_CHUNK, EROW), jnp.int32),
            (pltpu.VMEM((EROW, fc), jnp.float32),
             pltpu.VMEM((EROW, fc), jnp.float32)),
            pltpu.VMEM_SHARED((N_PAD, fc), jnp.float32),
            (pltpu.SemaphoreType.DMA, pltpu.SemaphoreType.DMA),
        ],
    )


_agg64 = _make_agg(64)
_agg128 = _make_agg(128)
_agg32 = _make_agg(32)


def _norm(col):
    return lax.rsqrt(jnp.maximum(col, 1.0))


def _dot(a, b):
    return lax.dot_general(a, b, (((1,), (0,)), ((), ())),
                           precision=lax.Precision.HIGHEST,
                           preferred_element_type=jnp.float32)


def _prep_body(x_ref, hist_ref, xs_ref):
    ns = _norm(hist_ref[1][:, :1])
    xs = x_ref[...] * ns
    xs_ref[0, :, :] = xs[:, :64]
    xs_ref[1, :, :] = xs[:, 64:]


_prep_kernel = pl.pallas_call(
    _prep_body,
    grid=(GRID_M,),
    in_specs=[
        pl.BlockSpec((BM, 128), lambda m: (m, 0)),
        pl.BlockSpec((NC, BM, 16), lambda m: (0, m, 0)),
    ],
    out_specs=pl.BlockSpec((NC, BM, 64), lambda m: (0, m, 0)),
    out_shape=jax.ShapeDtypeStruct((NC, N_PAD, 64), jnp.float32),
)


def _layer1_body(agg_ref, hist_ref, w_ref, b_ref, out_ref):
    a = jnp.concatenate([agg_ref[0], agg_ref[1]], axis=1)
    h = _dot(a, w_ref[...])
    nd = _norm(hist_ref[0][:, :1])
    ns = _norm(hist_ref[1][:, :1])
    g = ns * jnp.maximum(nd * h + b_ref[...], 0.0)
    out_ref[0, :, :] = g[:, :128]
    out_ref[1, :, :] = g[:, 128:]


_layer1_kernel = pl.pallas_call(
    _layer1_body,
    grid=(GRID_M,),
    in_specs=[
        pl.BlockSpec((NC, BM, 64), lambda m: (0, m, 0)),
        pl.BlockSpec((NC, BM, 16), lambda m: (0, m, 0)),
        pl.BlockSpec((128, 256), lambda m: (0, 0)),
        pl.BlockSpec((1, 256), lambda m: (0, 0)),
    ],
    out_specs=pl.BlockSpec((NC, BM, 128), lambda m: (0, m, 0)),
    out_shape=jax.ShapeDtypeStruct((NC, N_PAD, 128), jnp.float32),
)


def _layer2_body(agg_ref, hist_ref, w2_ref, b2_ref, w3_ref, out_ref):
    a = jnp.concatenate([agg_ref[0], agg_ref[1]], axis=1)
    h = _dot(a, w2_ref[...])
    nd = _norm(hist_ref[0][:, :1])
    ns = _norm(hist_ref[1][:, :1])
    g = ns * jnp.maximum(nd * h + b2_ref[...], 0.0)
    t = _dot(g, w3_ref[...])
    out_ref[0, :, :] = t[:, :32]
    out_ref[1, :, :] = t[:, 32:]


_layer2_kernel = pl.pallas_call(
    _layer2_body,
    grid=(GRID_M,),
    in_specs=[
        pl.BlockSpec((NC, BM, 128), lambda m: (0, m, 0)),
        pl.BlockSpec((NC, BM, 16), lambda m: (0, m, 0)),
        pl.BlockSpec((256, 256), lambda m: (0, 0)),
        pl.BlockSpec((1, 256), lambda m: (0, 0)),
        pl.BlockSpec((256, 64), lambda m: (0, 0)),
    ],
    out_specs=pl.BlockSpec((NC, BM, 32), lambda m: (0, m, 0)),
    out_shape=jax.ShapeDtypeStruct((NC, N_PAD, 32), jnp.float32),
)


def _final_body(agg_ref, hist_ref, b_ref, out_ref):
    a = jnp.concatenate([agg_ref[0], agg_ref[1]], axis=1)
    nd = _norm(hist_ref[0][:, :1])
    out_ref[...] = nd * a + b_ref[...]


_final_kernel = pl.pallas_call(
    _final_body,
    grid=(GRID_M,),
    in_specs=[
        pl.BlockSpec((NC, BM, 32), lambda m: (0, m, 0)),
        pl.BlockSpec((NC, BM, 16), lambda m: (0, m, 0)),
        pl.BlockSpec((1, 64), lambda m: (0, 0)),
    ],
    out_specs=pl.BlockSpec((BM, 64), lambda m: (m, 0)),
    out_shape=jax.ShapeDtypeStruct((N_PAD, 64), jnp.float32),
)


def kernel(features, edge_index, W1, b1, W2, b2, W3, b3):
    epad = jnp.full((2, EROWS * EROW - N_EDGES), PAD_IDX, jnp.int32)
    edges = jnp.concatenate([edge_index, epad], axis=1).reshape(2, EROWS, EROW)
    x = jnp.concatenate(
        [features, jnp.zeros((N_PAD - N_NODES, 128), jnp.float32)], axis=0)
    ones16 = jnp.ones((EROW, 16), jnp.float32)
    zeros16 = jnp.zeros((EROW, 16), jnp.float32)

    hist = _deg_kernel(edges, ones16, zeros16)
    xs = _prep_kernel(x, hist)
    a1 = _agg64(xs, edges, jnp.zeros((EROW, 64), jnp.float32))
    g1 = _layer1_kernel(a1, hist, W1, b1.reshape(1, -1))
    a2 = _agg128(g1, edges, jnp.zeros((EROW, 128), jnp.float32))
    t = _layer2_kernel(a2, hist, W2, b2.reshape(1, -1), W3)
    a3 = _agg32(t, edges, jnp.zeros((EROW, 32), jnp.float32))
    out = _final_kernel(a3, hist, b3.reshape(1, -1))
    return out[:N_NODES]


# speedup vs baseline: 6.8369x; 1.0197x over previous
import functools

import jax
import jax.numpy as jnp
from jax import lax
from jax.experimental import pallas as pl
from jax.experimental.pallas import tpu as pltpu, tpu_sc as plsc

N_NODES = 10000
N_PAD = 10240
N_EDGES = 320000
EROW = 128
EROWS = 2560
PAD_IDX = N_PAD - 1
NC, NS = 2, 16
ROWS_PER_TILE = EROWS // NS
STRIPE = N_PAD // NS
BM = 512
GRID_M = N_PAD // BM

_mesh = plsc.VectorSubcoreMesh(core_axis_name="c", subcore_axis_name="s",
                               num_cores=NC, num_subcores=NS)
_sc_params = pltpu.CompilerParams(use_tc_tiling_on_sc=False)


def _deg_body(edges, ones_h, zeros_h, hist, idx_v, ones_v, zbuf, hist_sp):
    c = lax.axis_index("c")
    s = lax.axis_index("s")
    base = s * ROWS_PER_TILE
    pltpu.sync_copy(edges.at[1 - c].at[pl.ds(base, ROWS_PER_TILE)], idx_v)
    pltpu.sync_copy(ones_h, ones_v)
    pltpu.sync_copy(zeros_h, zbuf)
    for k in range(STRIPE // EROW):
        pltpu.sync_copy(zbuf, hist_sp.at[pl.ds(s * STRIPE + k * EROW, EROW)])
    plsc.subcore_barrier()

    def body(r, carry):
        pltpu.sync_copy(ones_v, hist_sp.at[idx_v.at[r]], add=True)
        return carry

    lax.fori_loop(0, ROWS_PER_TILE, body, 0)
    plsc.subcore_barrier()
    pltpu.sync_copy(hist_sp.at[pl.ds(s * STRIPE, STRIPE)],
                    hist.at[c].at[pl.ds(s * STRIPE, STRIPE)])


_deg_kernel = pl.kernel(
    _deg_body,
    out_type=jax.ShapeDtypeStruct((NC, N_PAD, 16), jnp.float32),
    mesh=_mesh,
    compiler_params=_sc_params,
    scratch_types=[
        pltpu.VMEM((ROWS_PER_TILE, EROW), jnp.int32),
        pltpu.VMEM((EROW, 16), jnp.float32),
        pltpu.VMEM((EROW, 16), jnp.float32),
        pltpu.VMEM_SHARED((N_PAD, 16), jnp.float32),
    ],
)


IDX_CHUNK = 32
N_CHUNKS = ROWS_PER_TILE // IDX_CHUNK


def _agg_body(fc, depth, table, edges, zeros_h, agg,
              src_v, dst_v, gbuf, agg_sp, gsem):
    c = lax.axis_index("c")
    s = lax.axis_index("s")
    base = s * ROWS_PER_TILE
    pltpu.sync_copy(zeros_h, gbuf[0])
    for k in range(STRIPE // EROW):
        pltpu.sync_copy(gbuf[0], agg_sp.at[pl.ds(s * STRIPE + k * EROW, EROW)])
    plsc.subcore_barrier()

    tbl = table.at[c]
    for ci in range(N_CHUNKS):
        pltpu.sync_copy(edges.at[0].at[pl.ds(base + ci * IDX_CHUNK, IDX_CHUNK)],
                        src_v)
        pltpu.sync_copy(edges.at[1].at[pl.ds(base + ci * IDX_CHUNK, IDX_CHUNK)],
                        dst_v)
        for d in range(depth):
            pltpu.async_copy(tbl.at[src_v.at[d]], gbuf[d], gsem[d])

        def group(k, carry):
            r0 = depth * k
            for d in range(depth):
                pltpu.make_async_copy(tbl.at[src_v.at[r0 + d]], gbuf[d],
                                      gsem[d]).wait()
                pltpu.sync_copy(gbuf[d], agg_sp.at[dst_v.at[r0 + d]], add=True)
                pltpu.async_copy(tbl.at[src_v.at[r0 + d + depth]], gbuf[d],
                                 gsem[d])
            return carry

        lax.fori_loop(0, IDX_CHUNK // depth - 1, group, 0)
        for d in range(depth):
            r = IDX_CHUNK - depth + d
            pltpu.make_async_copy(tbl.at[src_v.at[r]], gbuf[d], gsem[d]).wait()
            pltpu.sync_copy(gbuf[d], agg_sp.at[dst_v.at[r]], add=True)
    plsc.subcore_barrier()
    pltpu.sync_copy(agg_sp.at[pl.ds(s * STRIPE, STRIPE)],
                    agg.at[c].at[pl.ds(s * STRIPE, STRIPE)])


def _make_agg(fc, depth):
    return pl.kernel(
        functools.partial(_agg_body, fc, depth),
        out_type=jax.ShapeDtypeStruct((NC, N_PAD, fc), jnp.float32),
        mesh=_mesh,
        compiler_params=_sc_params,
        scratch_types=[
            pltpu.VMEM((IDX_CHUNK, EROW), jnp.int32),
            pltpu.VMEM((IDX_CHUNK, EROW), jnp.int32),
            tuple(pltpu.VMEM((EROW, fc), jnp.float32) for _ in range(depth)),
            pltpu.VMEM_SHARED((N_PAD, fc), jnp.float32),
            tuple(pltpu.SemaphoreType.DMA for _ in range(depth)),
        ],
    )


_agg64 = _make_agg(64, 4)
_agg128 = _make_agg(128, 2)
_agg32 = _make_agg(32, 4)


def _norm(col):
    return lax.rsqrt(jnp.maximum(col, 1.0))


def _dot(a, b):
    return lax.dot_general(a, b, (((1,), (0,)), ((), ())),
                           precision=lax.Precision.HIGHEST,
                           preferred_element_type=jnp.float32)


def _prep_body(x_ref, hist_ref, xs_ref):
    ns = _norm(hist_ref[1][:, :1])
    xs = x_ref[...] * ns
    xs_ref[0, :, :] = xs[:, :64]
    xs_ref[1, :, :] = xs[:, 64:]


_prep_kernel = pl.pallas_call(
    _prep_body,
    grid=(GRID_M,),
    in_specs=[
        pl.BlockSpec((BM, 128), lambda m: (m, 0)),
        pl.BlockSpec((NC, BM, 16), lambda m: (0, m, 0)),
    ],
    out_specs=pl.BlockSpec((NC, BM, 64), lambda m: (0, m, 0)),
    out_shape=jax.ShapeDtypeStruct((NC, N_PAD, 64), jnp.float32),
)


def _layer1_body(agg_ref, hist_ref, w_ref, b_ref, out_ref):
    a = jnp.concatenate([agg_ref[0], agg_ref[1]], axis=1)
    h = _dot(a, w_ref[...])
    nd = _norm(hist_ref[0][:, :1])
    ns = _norm(hist_ref[1][:, :1])
    g = ns * jnp.maximum(nd * h + b_ref[...], 0.0)
    out_ref[0, :, :] = g[:, :128]
    out_ref[1, :, :] = g[:, 128:]


_layer1_kernel = pl.pallas_call(
    _layer1_body,
    grid=(GRID_M,),
    in_specs=[
        pl.BlockSpec((NC, BM, 64), lambda m: (0, m, 0)),
        pl.BlockSpec((NC, BM, 16), lambda m: (0, m, 0)),
        pl.BlockSpec((128, 256), lambda m: (0, 0)),
        pl.BlockSpec((1, 256), lambda m: (0, 0)),
    ],
    out_specs=pl.BlockSpec((NC, BM, 128), lambda m: (0, m, 0)),
    out_shape=jax.ShapeDtypeStruct((NC, N_PAD, 128), jnp.float32),
)


def _layer2_body(agg_ref, hist_ref, w2_ref, b2_ref, w3_ref, out_ref):
    a = jnp.concatenate([agg_ref[0], agg_ref[1]], axis=1)
    h = _dot(a, w2_ref[...])
    nd = _norm(hist_ref[0][:, :1])
    ns = _norm(hist_ref[1][:, :1])
    g = ns * jnp.maximum(nd * h + b2_ref[...], 0.0)
    t = _dot(g, w3_ref[...])
    out_ref[0, :, :] = t[:, :32]
    out_ref[1, :, :] = t[:, 32:]


_layer2_kernel = pl.pallas_call(
    _layer2_body,
    grid=(GRID_M,),
    in_specs=[
        pl.BlockSpec((NC, BM, 128), lambda m: (0, m, 0)),
        pl.BlockSpec((NC, BM, 16), lambda m: (0, m, 0)),
        pl.BlockSpec((256, 256), lambda m: (0, 0)),
        pl.BlockSpec((1, 256), lambda m: (0, 0)),
        pl.BlockSpec((256, 64), lambda m: (0, 0)),
    ],
    out_specs=pl.BlockSpec((NC, BM, 32), lambda m: (0, m, 0)),
    out_shape=jax.ShapeDtypeStruct((NC, N_PAD, 32), jnp.float32),
)


def _final_body(agg_ref, hist_ref, b_ref, out_ref):
    a = jnp.concatenate([agg_ref[0], agg_ref[1]], axis=1)
    nd = _norm(hist_ref[0][:, :1])
    out_ref[...] = nd * a + b_ref[...]


_final_kernel = pl.pallas_call(
    _final_body,
    grid=(GRID_M,),
    in_specs=[
        pl.BlockSpec((NC, BM, 32), lambda m: (0, m, 0)),
        pl.BlockSpec((NC, BM, 16), lambda m: (0, m, 0)),
        pl.BlockSpec((1, 64), lambda m: (0, 0)),
    ],
    out_specs=pl.BlockSpec((BM, 64), lambda m: (m, 0)),
    out_shape=jax.ShapeDtypeStruct((N_PAD, 64), jnp.float32),
)


def kernel(features, edge_index, W1, b1, W2, b2, W3, b3):
    epad = jnp.full((2, EROWS * EROW - N_EDGES), PAD_IDX, jnp.int32)
    edges = jnp.concatenate([edge_index, epad], axis=1).reshape(2, EROWS, EROW)
    x = jnp.concatenate(
        [features, jnp.zeros((N_PAD - N_NODES, 128), jnp.float32)], axis=0)
    ones16 = jnp.ones((EROW, 16), jnp.float32)
    zeros16 = jnp.zeros((EROW, 16), jnp.float32)

    hist = _deg_kernel(edges, ones16, zeros16)
    xs = _prep_kernel(x, hist)
    a1 = _agg64(xs, edges, jnp.zeros((EROW, 64), jnp.float32))
    g1 = _layer1_kernel(a1, hist, W1, b1.reshape(1, -1))
    a2 = _agg128(g1, edges, jnp.zeros((EROW, 128), jnp.float32))
    t = _layer2_kernel(a2, hist, W2, b2.reshape(1, -1), W3)
    a3 = _agg32(t, edges, jnp.zeros((EROW, 32), jnp.float32))
    out = _final_kernel(a3, hist, b3.reshape(1, -1))
    return out[:N_NODES]


# speedup vs baseline: 11.0361x; 1.6142x over previous
import functools

import jax
import jax.numpy as jnp
from jax import lax
from jax.experimental import pallas as pl
from jax.experimental.pallas import tpu as pltpu, tpu_sc as plsc

N_NODES = 10000
N_PAD = 10240
N_EDGES = 320000
EROW = 128
EROWS = 2560
PAD_IDX = N_PAD - 1
NC, NS = 2, 16
ROWS_PER_TILE = EROWS // NS
STRIPE = N_PAD // NS
BM = 512
GRID_M = N_PAD // BM

_mesh = plsc.VectorSubcoreMesh(core_axis_name="c", subcore_axis_name="s",
                               num_cores=NC, num_subcores=NS)
_sc_params = pltpu.CompilerParams(use_tc_tiling_on_sc=False)


def _deg_body(edges, ones_h, zeros_h, hist, idx_v, ones_v, zbuf, hist_sp):
    c = lax.axis_index("c")
    s = lax.axis_index("s")
    base = s * ROWS_PER_TILE
    pltpu.sync_copy(edges.at[1 - c].at[pl.ds(base, ROWS_PER_TILE)], idx_v)
    pltpu.sync_copy(ones_h, ones_v)
    pltpu.sync_copy(zeros_h, zbuf)
    for k in range(STRIPE // EROW):
        pltpu.sync_copy(zbuf, hist_sp.at[pl.ds(s * STRIPE + k * EROW, EROW)])
    plsc.subcore_barrier()

    def body(r, carry):
        pltpu.sync_copy(ones_v, hist_sp.at[idx_v.at[r]], add=True)
        return carry

    lax.fori_loop(0, ROWS_PER_TILE, body, 0)
    plsc.subcore_barrier()
    pltpu.sync_copy(hist_sp.at[pl.ds(s * STRIPE, STRIPE)],
                    hist.at[c].at[pl.ds(s * STRIPE, STRIPE)])


_deg_kernel = pl.kernel(
    _deg_body,
    out_type=jax.ShapeDtypeStruct((NC, N_PAD, 16), jnp.float32),
    mesh=_mesh,
    compiler_params=_sc_params,
    scratch_types=[
        pltpu.VMEM((ROWS_PER_TILE, EROW), jnp.int32),
        pltpu.VMEM((EROW, 16), jnp.float32),
        pltpu.VMEM((EROW, 16), jnp.float32),
        pltpu.VMEM_SHARED((N_PAD, 16), jnp.float32),
    ],
)


IDX_CHUNK = 32
N_CHUNKS = ROWS_PER_TILE // IDX_CHUNK


def _agg_body(fc, depth, table, edges, zeros_h, agg,
              src_v, dst_v, gbuf, tbl_sp, agg_sp, gsem):
    c = lax.axis_index("c")
    s = lax.axis_index("s")
    base = s * ROWS_PER_TILE
    pltpu.sync_copy(table.at[c].at[pl.ds(s * STRIPE, STRIPE)],
                    tbl_sp.at[pl.ds(s * STRIPE, STRIPE)])
    pltpu.sync_copy(zeros_h, gbuf[0])
    for k in range(STRIPE // EROW):
        pltpu.sync_copy(gbuf[0], agg_sp.at[pl.ds(s * STRIPE + k * EROW, EROW)])
    plsc.subcore_barrier()

    tbl = tbl_sp
    for ci in range(N_CHUNKS):
        pltpu.sync_copy(edges.at[0].at[pl.ds(base + ci * IDX_CHUNK, IDX_CHUNK)],
                        src_v)
        pltpu.sync_copy(edges.at[1].at[pl.ds(base + ci * IDX_CHUNK, IDX_CHUNK)],
                        dst_v)
        for d in range(depth):
            pltpu.async_copy(tbl.at[src_v.at[d]], gbuf[d], gsem[d])

        def group(k, carry):
            r0 = depth * k
            for d in range(depth):
                pltpu.make_async_copy(tbl.at[src_v.at[r0 + d]], gbuf[d],
                                      gsem[d]).wait()
                pltpu.sync_copy(gbuf[d], agg_sp.at[dst_v.at[r0 + d]], add=True)
                pltpu.async_copy(tbl.at[src_v.at[r0 + d + depth]], gbuf[d],
                                 gsem[d])
            return carry

        lax.fori_loop(0, IDX_CHUNK // depth - 1, group, 0)
        for d in range(depth):
            r = IDX_CHUNK - depth + d
            pltpu.make_async_copy(tbl.at[src_v.at[r]], gbuf[d], gsem[d]).wait()
            pltpu.sync_copy(gbuf[d], agg_sp.at[dst_v.at[r]], add=True)
    plsc.subcore_barrier()
    pltpu.sync_copy(agg_sp.at[pl.ds(s * STRIPE, STRIPE)],
                    agg.at[c].at[pl.ds(s * STRIPE, STRIPE)])


def _make_agg(fc, depth):
    return pl.kernel(
        functools.partial(_agg_body, fc, depth),
        out_type=jax.ShapeDtypeStruct((NC, N_PAD, fc), jnp.float32),
        mesh=_mesh,
        compiler_params=_sc_params,
        scratch_types=[
            pltpu.VMEM((IDX_CHUNK, EROW), jnp.int32),
            pltpu.VMEM((IDX_CHUNK, EROW), jnp.int32),
            tuple(pltpu.VMEM((EROW, fc), jnp.float32) for _ in range(depth)),
            pltpu.VMEM_SHARED((N_PAD, fc), jnp.float32),
            pltpu.VMEM_SHARED((N_PAD, fc), jnp.float32),
            tuple(pltpu.SemaphoreType.DMA for _ in range(depth)),
        ],
    )


_agg64 = _make_agg(64, 4)
_agg32 = _make_agg(32, 4)


def _norm(col):
    return lax.rsqrt(jnp.maximum(col, 1.0))


def _dot(a, b):
    return lax.dot_general(a, b, (((1,), (0,)), ((), ())),
                           precision=lax.Precision.HIGHEST,
                           preferred_element_type=jnp.float32)


def _prep_body(x_ref, hist_ref, xs_ref):
    ns = _norm(hist_ref[1][:, :1])
    xs = x_ref[...] * ns
    xs_ref[0, :, :] = xs[:, :64]
    xs_ref[1, :, :] = xs[:, 64:]


_prep_kernel = pl.pallas_call(
    _prep_body,
    grid=(GRID_M,),
    in_specs=[
        pl.BlockSpec((BM, 128), lambda m: (m, 0)),
        pl.BlockSpec((NC, BM, 16), lambda m: (0, m, 0)),
    ],
    out_specs=pl.BlockSpec((NC, BM, 64), lambda m: (0, m, 0)),
    out_shape=jax.ShapeDtypeStruct((NC, N_PAD, 64), jnp.float32),
)


def _layer1_body(agg_ref, hist_ref, w_ref, b_ref, out0_ref, out1_ref):
    a = jnp.concatenate([agg_ref[0], agg_ref[1]], axis=1)
    h = _dot(a, w_ref[...])
    nd = _norm(hist_ref[0][:, :1])
    ns = _norm(hist_ref[1][:, :1])
    g = ns * jnp.maximum(nd * h + b_ref[...], 0.0)
    out0_ref[0, :, :] = g[:, 0:64]
    out0_ref[1, :, :] = g[:, 64:128]
    out1_ref[0, :, :] = g[:, 128:192]
    out1_ref[1, :, :] = g[:, 192:256]


_layer1_kernel = pl.pallas_call(
    _layer1_body,
    grid=(GRID_M,),
    in_specs=[
        pl.BlockSpec((NC, BM, 64), lambda m: (0, m, 0)),
        pl.BlockSpec((NC, BM, 16), lambda m: (0, m, 0)),
        pl.BlockSpec((128, 256), lambda m: (0, 0)),
        pl.BlockSpec((1, 256), lambda m: (0, 0)),
    ],
    out_specs=[pl.BlockSpec((NC, BM, 64), lambda m: (0, m, 0)),
               pl.BlockSpec((NC, BM, 64), lambda m: (0, m, 0))],
    out_shape=[jax.ShapeDtypeStruct((NC, N_PAD, 64), jnp.float32),
               jax.ShapeDtypeStruct((NC, N_PAD, 64), jnp.float32)],
)


def _layer2_body(agg0_ref, agg1_ref, hist_ref, w2_ref, b2_ref, w3_ref,
                 out_ref):
    a = jnp.concatenate([agg0_ref[0], agg0_ref[1],
                         agg1_ref[0], agg1_ref[1]], axis=1)
    h = _dot(a, w2_ref[...])
    nd = _norm(hist_ref[0][:, :1])
    ns = _norm(hist_ref[1][:, :1])
    g = ns * jnp.maximum(nd * h + b2_ref[...], 0.0)
    t = _dot(g, w3_ref[...])
    out_ref[0, :, :] = t[:, :32]
    out_ref[1, :, :] = t[:, 32:]


_layer2_kernel = pl.pallas_call(
    _layer2_body,
    grid=(GRID_M,),
    in_specs=[
        pl.BlockSpec((NC, BM, 64), lambda m: (0, m, 0)),
        pl.BlockSpec((NC, BM, 64), lambda m: (0, m, 0)),
        pl.BlockSpec((NC, BM, 16), lambda m: (0, m, 0)),
        pl.BlockSpec((256, 256), lambda m: (0, 0)),
        pl.BlockSpec((1, 256), lambda m: (0, 0)),
        pl.BlockSpec((256, 64), lambda m: (0, 0)),
    ],
    out_specs=pl.BlockSpec((NC, BM, 32), lambda m: (0, m, 0)),
    out_shape=jax.ShapeDtypeStruct((NC, N_PAD, 32), jnp.float32),
)


def _final_body(agg_ref, hist_ref, b_ref, out_ref):
    a = jnp.concatenate([agg_ref[0], agg_ref[1]], axis=1)
    nd = _norm(hist_ref[0][:, :1])
    out_ref[...] = nd * a + b_ref[...]


_final_kernel = pl.pallas_call(
    _final_body,
    grid=(GRID_M,),
    in_specs=[
        pl.BlockSpec((NC, BM, 32), lambda m: (0, m, 0)),
        pl.BlockSpec((NC, BM, 16), lambda m: (0, m, 0)),
        pl.BlockSpec((1, 64), lambda m: (0, 0)),
    ],
    out_specs=pl.BlockSpec((BM, 64), lambda m: (m, 0)),
    out_shape=jax.ShapeDtypeStruct((N_PAD, 64), jnp.float32),
)


def kernel(features, edge_index, W1, b1, W2, b2, W3, b3):
    epad = jnp.full((2, EROWS * EROW - N_EDGES), PAD_IDX, jnp.int32)
    edges = jnp.concatenate([edge_index, epad], axis=1).reshape(2, EROWS, EROW)
    x = jnp.concatenate(
        [features, jnp.zeros((N_PAD - N_NODES, 128), jnp.float32)], axis=0)
    ones16 = jnp.ones((EROW, 16), jnp.float32)
    zeros16 = jnp.zeros((EROW, 16), jnp.float32)

    z64 = jnp.zeros((EROW, 64), jnp.float32)
    hist = _deg_kernel(edges, ones16, zeros16)
    xs = _prep_kernel(x, hist)
    a1 = _agg64(xs, edges, z64)
    t0, t1 = _layer1_kernel(a1, hist, W1, b1.reshape(1, -1))
    a2_0 = _agg64(t0, edges, z64)
    a2_1 = _agg64(t1, edges, z64)
    t = _layer2_kernel(a2_0, a2_1, hist, W2, b2.reshape(1, -1), W3)
    a3 = _agg32(t, edges, jnp.zeros((EROW, 32), jnp.float32))
    out = _final_kernel(a3, hist, b3.reshape(1, -1))
    return out[:N_NODES]
